# Initial kernel scaffold; baseline (speedup 1.0000x reference)
import functools

import jax
import jax.numpy as jnp
import numpy as np
from jax import lax
from jax.experimental import pallas as pl
from jax.experimental.pallas import tpu as pltpu
from jax.experimental.pallas import tpu_sc as plsc

N_NODES = 10000
N_EDGES = 320000
EMB = 128

NC = 2
NS = 16
L = 16

GROUP = 128
G_PER_TILE = 79
E_PAD = NC * NS * G_PER_TILE * GROUP
N_PAD = 10240
ROWS_PER_TILE = N_PAD // NS
N_COMBO = 32


def _sc_body(nf, src, dst, cmb, agg_out, cnt_out,
             src_v, dst_v, c_v, rows_v, stage_v, agg_sp, cnt_sp, sem):
    cid = lax.axis_index("c")
    sid = lax.axis_index("s")
    tile = cid * NS + sid

    zeros16 = jnp.zeros((L,), jnp.float32)
    ones16 = jnp.ones((L,), jnp.float32)
    iota16 = lax.iota(jnp.int32, L)

    def zrow(i, carry):
        for j in range(EMB // L):
            rows_v[i, pl.ds(j * L, L)] = zeros16
        for j in range(N_COMBO // L):
            stage_v[i, pl.ds(j * L, L)] = zeros16
        return carry

    lax.fori_loop(0, GROUP, zrow, 0)
    base = sid * ROWS_PER_TILE
    for i in range(ROWS_PER_TILE // GROUP):
        pltpu.sync_copy(rows_v, agg_sp.at[pl.ds(base + i * GROUP, GROUP)])
        pltpu.sync_copy(stage_v, cnt_sp.at[pl.ds(base + i * GROUP, GROUP)])
    plsc.subcore_barrier()

    pltpu.sync_copy(src.at[tile], src_v)
    pltpu.sync_copy(dst.at[tile], dst_v)
    pltpu.sync_copy(cmb.at[tile], c_v)

    def body(g, carry):
        pltpu.async_copy(nf.at[src_v.at[g]], rows_v, sem).wait()
        for sub in range(GROUP // L):
            f = c_v[g, pl.ds(sub * L, L)]
            plsc.store_scatter(stage_v, [iota16 + sub * L, f], ones16)
        pltpu.sync_copy(rows_v, agg_sp.at[dst_v.at[g]], add=True)
        pltpu.sync_copy(stage_v, cnt_sp.at[dst_v.at[g]], add=True)
        for sub in range(GROUP // L):
            f = c_v[g, pl.ds(sub * L, L)]
            plsc.store_scatter(stage_v, [iota16 + sub * L, f], zeros16)
        return carry

    lax.fori_loop(0, G_PER_TILE, body, 0)
    plsc.subcore_barrier()

    for i in range(ROWS_PER_TILE // GROUP):
        r0 = base + i * GROUP
        pltpu.sync_copy(agg_sp.at[pl.ds(r0, GROUP)], agg_out.at[cid, pl.ds(r0, GROUP)])
        pltpu.sync_copy(cnt_sp.at[pl.ds(r0, GROUP)], cnt_out.at[cid, pl.ds(r0, GROUP)])


def _sc_scatter(node_feats, src3, dst3, c3):
    mesh = plsc.VectorSubcoreMesh(core_axis_name="c", subcore_axis_name="s")
    f32 = jnp.float32
    return pl.kernel(
        _sc_body,
        out_type=[
            jax.ShapeDtypeStruct((NC, N_PAD, EMB), f32),
            jax.ShapeDtypeStruct((NC, N_PAD, N_COMBO), f32),
        ],
        mesh=mesh,
        scratch_types=[
            pltpu.VMEM((G_PER_TILE, GROUP), jnp.int32),
            pltpu.VMEM((G_PER_TILE, GROUP), jnp.int32),
            pltpu.VMEM((G_PER_TILE, GROUP), jnp.int32),
            pltpu.VMEM((GROUP, EMB), f32),
            pltpu.VMEM((GROUP, N_COMBO), f32),
            pltpu.VMEM_SHARED((N_PAD, EMB), f32),
            pltpu.VMEM_SHARED((N_PAD, N_COMBO), f32),
            pltpu.SemaphoreType.DMA,
        ],
    )(node_feats, src3, dst3, c3)


def _tc_body(aggp, cntp, w0t, w1t_e, m0, m1, dmask, bsum,
             w1t, b1, w2t, b2, gamma, beta, out):
    te = (jnp.dot(m0[...], w0t[...], preferred_element_type=jnp.float32)
          + jnp.dot(m1[...], w1t_e[...], preferred_element_type=jnp.float32)
          + dmask[...] * bsum[...])
    cnt = cntp[0] + cntp[1]
    agg = (aggp[0] + aggp[1]
           + jnp.dot(cnt, te, preferred_element_type=jnp.float32))
    h1 = jnp.maximum(
        jnp.dot(agg, w1t[...], preferred_element_type=jnp.float32) + b1[...], 0.0)
    h2 = jnp.dot(h1, w2t[...], preferred_element_type=jnp.float32) + b2[...]
    rows = lax.broadcasted_iota(jnp.int32, (N_PAD, 1), 0)
    m = (rows < N_NODES).astype(jnp.float32)
    inv_n = 1.0 / N_NODES
    mean = jnp.sum(h2 * m, axis=0, keepdims=True) * inv_n
    d = (h2 - mean) * m
    var = jnp.sum(d * d, axis=0, keepdims=True) * inv_n
    out[...] = (h2 - mean) * lax.rsqrt(var + 1e-5) * gamma[...] + beta[...]


def kernel(node_feats, edge_index, edge_feat_0, edge_feat_1,
           We0, be0, We1, be1, W1, b1, W2, b2, gamma, beta):
    src = edge_index[0].astype(jnp.int32)
    dst = edge_index[1].astype(jnp.int32)
    cmb = (edge_feat_0 * 3 + edge_feat_1).astype(jnp.int32)

    pad = E_PAD - N_EDGES
    ar = jnp.arange(pad, dtype=jnp.int32)
    src_p = jnp.concatenate([src, ar % N_NODES]).reshape(NC * NS, G_PER_TILE, GROUP)
    dst_p = jnp.concatenate([dst, N_NODES + ar % (N_PAD - N_NODES)]
                            ).reshape(NC * NS, G_PER_TILE, GROUP)
    cmb_p = jnp.concatenate([cmb, 18 + ar % (N_COMBO - 18)]
                            ).reshape(NC * NS, G_PER_TILE, GROUP)

    agg_parts, cnt_parts = _sc_scatter(node_feats, src_p, dst_p, cmb_p)

    m0 = np.zeros((N_COMBO, 6), np.float32)
    m1 = np.zeros((N_COMBO, 3), np.float32)
    dm = np.zeros((N_COMBO, 1), np.float32)
    for k in range(18):
        m0[k, k // 3] = 1.0
        m1[k, k % 3] = 1.0
        dm[k, 0] = 1.0

    out = pl.pallas_call(
        _tc_body,
        out_shape=jax.ShapeDtypeStruct((N_PAD, EMB), jnp.float32),
    )(agg_parts, cnt_parts,
      We0.T, We1.T, jnp.asarray(m0), jnp.asarray(m1), jnp.asarray(dm),
      (be0 + be1).reshape(1, EMB),
      W1.T, b1.reshape(1, -1), W2.T, b2.reshape(1, -1),
      gamma.reshape(1, EMB), beta.reshape(1, EMB))
    return out[:N_NODES]


# speedup vs baseline: 10.3842x; 10.3842x over previous
import jax
import jax.numpy as jnp
import numpy as np
from jax import lax
from jax.experimental import pallas as pl
from jax.experimental.pallas import tpu as pltpu
from jax.experimental.pallas import tpu_sc as plsc

N_NODES = 10000
N_EDGES = 320000
EMB = 128

NC = 2
NS = 16

GROUP = 128
G_PER_TILE = 79
E_PAD = NC * NS * G_PER_TILE * GROUP
N_PAD = 10240
ROWS_PER_TILE = N_PAD // NS
N_COMBO = 32
TE_REP = 64


def _sc_body(nf, te, src, dst, tix, agg_out,
             src_v, dst_v, t_v, rows_v, agg_sp, sem):
    cid = lax.axis_index("c")
    sid = lax.axis_index("s")
    tile = cid * NS + sid

    zeros16 = jnp.zeros((16,), jnp.float32)

    def zrow(i, carry):
        for j in range(EMB // 16):
            rows_v[i, pl.ds(j * 16, 16)] = zeros16
        return carry

    lax.fori_loop(0, GROUP, zrow, 0)
    base = sid * ROWS_PER_TILE
    for i in range(ROWS_PER_TILE // GROUP):
        pltpu.sync_copy(rows_v, agg_sp.at[pl.ds(base + i * GROUP, GROUP)])
    plsc.subcore_barrier()

    pltpu.sync_copy(src.at[tile], src_v)
    pltpu.sync_copy(dst.at[tile], dst_v)
    pltpu.sync_copy(tix.at[tile], t_v)

    def body(g, carry):
        pltpu.async_copy(nf.at[src_v.at[g]], rows_v, sem).wait()
        pltpu.async_copy(te.at[t_v.at[g]], rows_v, sem, add=True).wait()
        pltpu.sync_copy(rows_v, agg_sp.at[dst_v.at[g]], add=True)
        return carry

    lax.fori_loop(0, G_PER_TILE, body, 0)
    plsc.subcore_barrier()

    for i in range(ROWS_PER_TILE // GROUP):
        r0 = base + i * GROUP
        pltpu.sync_copy(agg_sp.at[pl.ds(r0, GROUP)], agg_out.at[cid, pl.ds(r0, GROUP)])


def _sc_scatter(node_feats, te_rep, src3, dst3, tix3):
    mesh = plsc.VectorSubcoreMesh(core_axis_name="c", subcore_axis_name="s")
    f32 = jnp.float32
    return pl.kernel(
        _sc_body,
        out_type=[jax.ShapeDtypeStruct((NC, N_PAD, EMB), f32)],
        mesh=mesh,
        scratch_types=[
            pltpu.VMEM((G_PER_TILE, GROUP), jnp.int32),
            pltpu.VMEM((G_PER_TILE, GROUP), jnp.int32),
            pltpu.VMEM((G_PER_TILE, GROUP), jnp.int32),
            pltpu.VMEM((GROUP, EMB), f32),
            pltpu.VMEM_SHARED((N_PAD, EMB), f32),
            pltpu.SemaphoreType.DMA,
        ],
        compiler_params=pltpu.CompilerParams(needs_layout_passes=False),
    )(node_feats, te_rep, src3, dst3, tix3)


def _tc_body(aggp, w1t, b1, w2t, b2, gamma, beta, out):
    agg = aggp[0] + aggp[1]
    h1 = jnp.maximum(
        jnp.dot(agg, w1t[...], preferred_element_type=jnp.float32) + b1[...], 0.0)
    h2 = jnp.dot(h1, w2t[...], preferred_element_type=jnp.float32) + b2[...]
    rows = lax.broadcasted_iota(jnp.int32, (N_PAD, 1), 0)
    m = (rows < N_NODES).astype(jnp.float32)
    inv_n = 1.0 / N_NODES
    mean = jnp.sum(h2 * m, axis=0, keepdims=True) * inv_n
    d = (h2 - mean) * m
    var = jnp.sum(d * d, axis=0, keepdims=True) * inv_n
    out[...] = (h2 - mean) * lax.rsqrt(var + 1e-5) * gamma[...] + beta[...]


def kernel(node_feats, edge_index, edge_feat_0, edge_feat_1,
           We0, be0, We1, be1, W1, b1, W2, b2, gamma, beta):
    src = edge_index[0].astype(jnp.int32)
    dst = edge_index[1].astype(jnp.int32)
    cmb = (edge_feat_0 * 3 + edge_feat_1).astype(jnp.int32)

    i0 = np.arange(18) // 3
    i1 = np.arange(18) % 3
    te18 = We0.T[i0] + We1.T[i1] + (be0 + be1)[None, :]
    te32 = jnp.concatenate([te18, jnp.zeros((N_COMBO - 18, EMB), jnp.float32)])
    te_rep = jnp.tile(te32, (TE_REP, 1))

    pad = E_PAD - N_EDGES
    ar = jnp.arange(pad, dtype=jnp.int32)
    arE = jnp.arange(E_PAD, dtype=jnp.int32)
    src_p = jnp.concatenate([src, ar % N_NODES]).reshape(NC * NS, G_PER_TILE, GROUP)
    dst_p = jnp.concatenate([dst, N_NODES + ar % (N_PAD - N_NODES)]
                            ).reshape(NC * NS, G_PER_TILE, GROUP)
    cmb_p = jnp.concatenate([cmb, 18 + ar % (N_COMBO - 18)])
    tix_p = (cmb_p + N_COMBO * (arE % TE_REP)).reshape(NC * NS, G_PER_TILE, GROUP)

    (agg_parts,) = _sc_scatter(node_feats, te_rep, src_p, dst_p, tix_p)

    out = pl.pallas_call(
        _tc_body,
        out_shape=jax.ShapeDtypeStruct((N_PAD, EMB), jnp.float32),
    )(agg_parts, W1.T, b1.reshape(1, -1), W2.T, b2.reshape(1, -1),
      gamma.reshape(1, EMB), beta.reshape(1, EMB))
    return out[:N_NODES]


# speedup vs baseline: 15.4216x; 1.4851x over previous
import jax
import jax.numpy as jnp
import numpy as np
from jax import lax
from jax.experimental import pallas as pl
from jax.experimental.pallas import tpu as pltpu
from jax.experimental.pallas import tpu_sc as plsc

N_NODES = 10000
N_EDGES = 320000
EMB = 128

NC = 2
NS = 16

GROUP = 96
CHUNK = 15
N_CHUNKS = 7
G_PER_TILE = CHUNK * N_CHUNKS
E_PAD = NC * NS * G_PER_TILE * GROUP
N_PAD = 10240
ROWS_PER_TILE = N_PAD // NS
N_COMBO = 32
TE_REP = 64


def _sc_body(nf, te, src, dst, tix, agg_out,
             src_v, dst_v, t_v, rows_v, agg_sp, gsem, tsem, ssem, isem):
    cid = lax.axis_index("c")
    sid = lax.axis_index("s")
    tile = cid * NS + sid

    zeros16 = jnp.zeros((16,), jnp.float32)

    def zrow(i, carry):
        for j in range(EMB // 16):
            rows_v[0, i, pl.ds(j * 16, 16)] = zeros16
        return carry

    lax.fori_loop(0, 64, zrow, 0)
    base = sid * ROWS_PER_TILE
    for i in range(ROWS_PER_TILE // 64):
        pltpu.sync_copy(rows_v.at[0, pl.ds(0, 64)],
                        agg_sp.at[pl.ds(base + i * 64, 64)])
    plsc.subcore_barrier()

    G = G_PER_TILE

    def pw(g):
        c = lax.div(g, CHUNK)
        return lax.rem(c, 2), lax.rem(g, CHUNK)

    def start_idx_load(c):
        p = lax.rem(c, 2)
        pltpu.async_copy(src.at[tile, c], src_v.at[p], isem)
        pltpu.async_copy(dst.at[tile, c], dst_v.at[p], isem)
        pltpu.async_copy(tix.at[tile, c], t_v.at[p], isem)

    def wait_idx_load(c):
        p = lax.rem(c, 2)
        pltpu.make_async_copy(src.at[tile, c], src_v.at[p], isem).wait()
        pltpu.make_async_copy(dst.at[tile, c], dst_v.at[p], isem).wait()
        pltpu.make_async_copy(tix.at[tile, c], t_v.at[p], isem).wait()

    def start_gather(g):
        p, w = pw(g)
        pltpu.async_copy(nf.at[src_v.at[p, w]], rows_v.at[lax.rem(g, 3)], gsem)

    def wait_gather(g):
        p, w = pw(g)
        pltpu.make_async_copy(nf.at[src_v.at[p, w]], rows_v.at[lax.rem(g, 3)],
                              gsem).wait()

    def start_te(g):
        p, w = pw(g)
        pltpu.async_copy(te.at[t_v.at[p, w]], rows_v.at[lax.rem(g, 3)], tsem,
                         add=True)

    def wait_te(g):
        p, w = pw(g)
        pltpu.make_async_copy(te.at[t_v.at[p, w]], rows_v.at[lax.rem(g, 3)],
                              tsem).wait()

    def start_scatter(g):
        p, w = pw(g)
        pltpu.async_copy(rows_v.at[lax.rem(g, 3)], agg_sp.at[dst_v.at[p, w]],
                         ssem, add=True)

    def wait_scatter(g):
        p, w = pw(g)
        pltpu.make_async_copy(rows_v.at[lax.rem(g, 3)], agg_sp.at[dst_v.at[p, w]],
                              ssem).wait()

    start_idx_load(0)
    wait_idx_load(0)
    start_gather(0)

    def body(g, carry):
        @pl.when(jnp.logical_and(lax.rem(g, CHUNK) == 2,
                                 lax.div(g, CHUNK) <= N_CHUNKS - 2))
        def _():
            start_idx_load(lax.div(g, CHUNK) + 1)

        @pl.when(jnp.logical_and(lax.rem(g, CHUNK) == 0,
                                 jnp.logical_and(g >= CHUNK, g <= G - 1)))
        def _():
            wait_idx_load(lax.div(g, CHUNK))

        @pl.when(jnp.logical_and(g >= 2, g <= G + 1))
        def _():
            wait_scatter(g - 2)

        @pl.when(g <= G - 2)
        def _():
            start_gather(g + 1)

        @pl.when(g <= G - 1)
        def _():
            wait_gather(g)
            start_te(g)

        @pl.when(jnp.logical_and(g >= 1, g <= G))
        def _():
            wait_te(g - 1)
            start_scatter(g - 1)

        return carry

    lax.fori_loop(0, G + 2, body, 0)
    plsc.subcore_barrier()

    for i in range(ROWS_PER_TILE // 128):
        r0 = base + i * 128
        pltpu.sync_copy(agg_sp.at[pl.ds(r0, 128)], agg_out.at[cid, pl.ds(r0, 128)])


def _sc_scatter(node_feats, te_rep, src4, dst4, tix4):
    mesh = plsc.VectorSubcoreMesh(core_axis_name="c", subcore_axis_name="s")
    f32 = jnp.float32
    return pl.kernel(
        _sc_body,
        out_type=[jax.ShapeDtypeStruct((NC, N_PAD, EMB), f32)],
        mesh=mesh,
        scratch_types=[
            pltpu.VMEM((2, CHUNK, GROUP), jnp.int32),
            pltpu.VMEM((2, CHUNK, GROUP), jnp.int32),
            pltpu.VMEM((2, CHUNK, GROUP), jnp.int32),
            pltpu.VMEM((3, GROUP, EMB), f32),
            pltpu.VMEM_SHARED((N_PAD, EMB), f32),
            pltpu.SemaphoreType.DMA,
            pltpu.SemaphoreType.DMA,
            pltpu.SemaphoreType.DMA,
            pltpu.SemaphoreType.DMA,
        ],
        compiler_params=pltpu.CompilerParams(needs_layout_passes=False),
    )(node_feats, te_rep, src4, dst4, tix4)


def _tc_body(aggp, w1t, b1, w2t, b2, gamma, beta, out):
    agg = aggp[0] + aggp[1]
    h1 = jnp.maximum(
        jnp.dot(agg, w1t[...], preferred_element_type=jnp.float32) + b1[...], 0.0)
    h2 = jnp.dot(h1, w2t[...], preferred_element_type=jnp.float32) + b2[...]
    rows = lax.broadcasted_iota(jnp.int32, (N_PAD, 1), 0)
    m = (rows < N_NODES).astype(jnp.float32)
    inv_n = 1.0 / N_NODES
    mean = jnp.sum(h2 * m, axis=0, keepdims=True) * inv_n
    d = (h2 - mean) * m
    var = jnp.sum(d * d, axis=0, keepdims=True) * inv_n
    out[...] = (h2 - mean) * lax.rsqrt(var + 1e-5) * gamma[...] + beta[...]


def kernel(node_feats, edge_index, edge_feat_0, edge_feat_1,
           We0, be0, We1, be1, W1, b1, W2, b2, gamma, beta):
    src = edge_index[0].astype(jnp.int32)
    dst = edge_index[1].astype(jnp.int32)
    cmb = (edge_feat_0 * 3 + edge_feat_1).astype(jnp.int32)

    i0 = np.arange(18) // 3
    i1 = np.arange(18) % 3
    te18 = We0.T[i0] + We1.T[i1] + (be0 + be1)[None, :]
    te32 = jnp.concatenate([te18, jnp.zeros((N_COMBO - 18, EMB), jnp.float32)])
    te_rep = jnp.tile(te32, (TE_REP, 1))

    pad = E_PAD - N_EDGES
    ar = jnp.arange(pad, dtype=jnp.int32)
    arE = jnp.arange(E_PAD, dtype=jnp.int32)
    shp = (NC * NS, N_CHUNKS, CHUNK, GROUP)
    src_p = jnp.concatenate([src, ar % N_NODES]).reshape(shp)
    dst_p = jnp.concatenate([dst, N_NODES + ar % (N_PAD - N_NODES)]).reshape(shp)
    cmb_p = jnp.concatenate([cmb, 18 + ar % (N_COMBO - 18)])
    tix_p = (cmb_p + N_COMBO * (arE % TE_REP)).reshape(shp)

    (agg_parts,) = _sc_scatter(node_feats, te_rep, src_p, dst_p, tix_p)

    out = pl.pallas_call(
        _tc_body,
        out_shape=jax.ShapeDtypeStruct((N_PAD, EMB), jnp.float32),
    )(agg_parts, W1.T, b1.reshape(1, -1), W2.T, b2.reshape(1, -1),
      gamma.reshape(1, EMB), beta.reshape(1, EMB))
    return out[:N_NODES]
